# baseline (speedup 1.0000x reference)
import functools

import jax
import jax.numpy as jnp
from jax.experimental import pallas as pl
from jax.experimental.pallas import tpu as pltpu


def _fused(x_ref, w_ref, embc_ref, bc_ref, onehot_ref, out_ref,
           state_acc, logits_acc, *, s_len, n_s, n_w):
    t = pl.program_id(0)

    @pl.when(t == 0)
    def _init():
        state_acc[...] = jnp.sum(x_ref[...], axis=1)
        logits_acc[...] = jnp.zeros_like(logits_acc)

    @pl.when((t > 0) & (t < n_s))
    def _sum_slice():
        state_acc[...] += jnp.sum(x_ref[...], axis=1)

    @pl.when(t == n_s)
    def _normalize():
        state_acc[...] *= (1.0 / s_len)

    @pl.when(t >= n_s)
    def _w_phase():
        proj_rows = jax.lax.dot_general(
            state_acc[...], w_ref[...],
            dimension_numbers=(((1,), (1,)), ((), ())),
            preferred_element_type=jnp.float32)
        logits_acc[...] += jax.lax.dot_general(
            proj_rows + bc_ref[...], embc_ref[...],
            dimension_numbers=(((1,), (1,)), ((), ())),
            preferred_element_type=jnp.float32)

    @pl.when(t == n_s + n_w - 1)
    def _finish():
        logits = logits_acc[...]
        m = jnp.max(logits, axis=1, keepdims=True)
        lse = jnp.log(jnp.sum(jnp.exp(logits - m), axis=1, keepdims=True)) + m
        logp = logits - lse
        picked = jnp.sum(logp * onehot_ref[...], axis=1, keepdims=True)
        out_ref[...] = jnp.sum(picked, axis=0, keepdims=True) * (
            -0.001 / logits.shape[0])


def kernel(hidden_states, representations, W, b, current_indices,
           current_expert_idx, current_depth):
    B, S, H = hidden_states.shape
    E = representations.shape[0]
    SB = 64
    RW = 256
    n_s = S // SB
    n_w = H // RW
    T = n_s + n_w

    emb = jnp.take(representations, current_indices, axis=0)
    onehot = (jax.lax.iota(jnp.int32, E)[None, :]
              == jnp.asarray(current_expert_idx, jnp.int32)).astype(jnp.float32)
    b2 = b.reshape(1, H)

    def h_map(t):
        return (0, jnp.minimum(t, n_s - 1), 0)

    def w_map(t):
        return (jnp.clip(t - n_s, 0, n_w - 1), 0)

    def wc_map(t):
        return (0, jnp.clip(t - n_s, 0, n_w - 1))

    out = pl.pallas_call(
        functools.partial(_fused, s_len=S, n_s=n_s, n_w=n_w),
        grid=(T,),
        in_specs=[
            pl.BlockSpec((B, SB, H), h_map),
            pl.BlockSpec((RW, H), w_map),
            pl.BlockSpec((E, RW), wc_map),
            pl.BlockSpec((1, RW), wc_map),
            pl.BlockSpec((1, E), lambda t: (0, 0)),
        ],
        out_specs=pl.BlockSpec((1, 1), lambda t: (0, 0)),
        out_shape=jax.ShapeDtypeStruct((1, 1), jnp.float32),
        scratch_shapes=[pltpu.VMEM((B, H), jnp.float32),
                        pltpu.VMEM((B, E), jnp.float32)],
    )(hidden_states, W, emb, b2, onehot)
    return out[0, 0]


# speedup vs baseline: 1.1046x; 1.1046x over previous
import functools

import jax
import jax.numpy as jnp
from jax.experimental import pallas as pl
from jax.experimental.pallas import tpu as pltpu


def _fused(x_ref, w_ref, emb_ref, b_ref, onehot_ref, out_ref, acc_ref, *,
           s_len, n_chunks):
    k = pl.program_id(0)

    @pl.when(k == 0)
    def _init():
        acc_ref[...] = jnp.zeros_like(acc_ref)

    state_chunk = jnp.sum(x_ref[...], axis=1) * (1.0 / s_len)
    acc_ref[...] += jax.lax.dot_general(
        state_chunk, w_ref[...],
        dimension_numbers=(((1,), (1,)), ((), ())),
        preferred_element_type=jnp.float32)

    @pl.when(k == n_chunks - 1)
    def _finish():
        proj = acc_ref[...] + b_ref[...]
        logits = jax.lax.dot_general(
            proj, emb_ref[...],
            dimension_numbers=(((1,), (1,)), ((), ())),
            preferred_element_type=jnp.float32)
        m = jnp.max(logits, axis=1, keepdims=True)
        lse = jnp.log(jnp.sum(jnp.exp(logits - m), axis=1, keepdims=True)) + m
        logp = logits - lse
        picked = jnp.sum(logp * onehot_ref[...], axis=1, keepdims=True)
        out_ref[...] = jnp.sum(picked, axis=0, keepdims=True) * (-0.001 / logits.shape[0])


def kernel(hidden_states, representations, W, b, current_indices,
           current_expert_idx, current_depth):
    B, S, H = hidden_states.shape
    E = representations.shape[0]
    C = 128
    n = H // C

    emb = jnp.take(representations, current_indices, axis=0)
    onehot = (jax.lax.iota(jnp.int32, E)[None, :]
              == jnp.asarray(current_expert_idx, jnp.int32)).astype(jnp.float32)
    b2 = b.reshape(1, H)

    out = pl.pallas_call(
        functools.partial(_fused, s_len=S, n_chunks=n),
        grid=(n,),
        in_specs=[
            pl.BlockSpec((B, S, C), lambda k: (0, 0, k)),
            pl.BlockSpec((H, C), lambda k: (0, k)),
            pl.BlockSpec((E, H), lambda k: (0, 0)),
            pl.BlockSpec((1, H), lambda k: (0, 0)),
            pl.BlockSpec((1, E), lambda k: (0, 0)),
        ],
        out_specs=pl.BlockSpec((1, 1), lambda k: (0, 0)),
        out_shape=jax.ShapeDtypeStruct((1, 1), jnp.float32),
        scratch_shapes=[pltpu.VMEM((B, H), jnp.float32)],
    )(hidden_states, W, emb, b2, onehot)
    return out[0, 0]


# speedup vs baseline: 1.2063x; 1.0920x over previous
import functools

import jax
import jax.numpy as jnp
from jax.experimental import pallas as pl
from jax.experimental.pallas import tpu as pltpu


def _fused(x_ref, w_ref, rep_ref, b_ref, idx_ref, tgt_ref, out_ref, acc_ref,
           *, s_len, n_chunks):
    k = pl.program_id(0)

    @pl.when(k == 0)
    def _init():
        acc_ref[...] = jnp.zeros_like(acc_ref)

    state_chunk = jnp.sum(x_ref[...], axis=1) * (1.0 / s_len)
    acc_ref[...] += jax.lax.dot_general(
        state_chunk, w_ref[...],
        dimension_numbers=(((1,), (1,)), ((), ())),
        preferred_element_type=jnp.float32)

    @pl.when(k == n_chunks - 1)
    def _finish():
        proj = acc_ref[...] + b_ref[...]
        logits_full = jax.lax.dot_general(
            proj, rep_ref[...],
            dimension_numbers=(((1,), (1,)), ((), ())),
            preferred_element_type=jnp.float32)
        n_rows = logits_full.shape[1]
        n_e = idx_ref.shape[1]
        perm = (jax.lax.broadcasted_iota(jnp.int32, (n_rows, n_e), 0)
                == idx_ref[...]).astype(jnp.float32)
        logits = jax.lax.dot_general(
            logits_full, perm,
            dimension_numbers=(((1,), (0,)), ((), ())),
            preferred_element_type=jnp.float32)
        m = jnp.max(logits, axis=1, keepdims=True)
        lse = jnp.log(jnp.sum(jnp.exp(logits - m), axis=1, keepdims=True)) + m
        logp = logits - lse
        onehot = (jax.lax.broadcasted_iota(jnp.int32, (1, n_e), 1)
                  == tgt_ref[...]).astype(jnp.float32)
        picked = jnp.sum(logp * onehot, axis=1, keepdims=True)
        out_ref[...] = jnp.sum(picked, axis=0, keepdims=True) * (
            -0.001 / logits.shape[0])


def kernel(hidden_states, representations, W, b, current_indices,
           current_expert_idx, current_depth):
    B, S, H = hidden_states.shape
    E = current_indices.shape[0]
    C = 256
    n = H // C

    idx2d = current_indices.astype(jnp.int32).reshape(1, E)
    tgt = jnp.asarray(current_expert_idx, jnp.int32).reshape(1, 1)
    b2 = b.reshape(1, H)

    out = pl.pallas_call(
        functools.partial(_fused, s_len=S, n_chunks=n),
        grid=(n,),
        in_specs=[
            pl.BlockSpec((B, S, C), lambda k: (0, 0, k)),
            pl.BlockSpec((H, C), lambda k: (0, k)),
            pl.BlockSpec(representations.shape, lambda k: (0, 0)),
            pl.BlockSpec((1, H), lambda k: (0, 0)),
            pl.BlockSpec((1, E), lambda k: (0, 0)),
            pl.BlockSpec((1, 1), lambda k: (0, 0)),
        ],
        out_specs=pl.BlockSpec((1, 1), lambda k: (0, 0)),
        out_shape=jax.ShapeDtypeStruct((1, 1), jnp.float32),
        scratch_shapes=[pltpu.VMEM((B, H), jnp.float32)],
    )(hidden_states, W, representations, b2, idx2d, tgt)
    return out[0, 0]
